# baseline (speedup 1.0000x reference)
import jax
import jax.numpy as jnp
from jax.experimental import pallas as pl
from jax.experimental.pallas import tpu as pltpu

ALPHA_F, GAMMA_F = 0.25, 1.0
ALPHA_R, GAMMA_R, BETA_R = 0.5, 1.5, 0.11
VAR0, VAR1 = 0.1, 0.2
_B = 2.718281828459045 ** (GAMMA_R / ALPHA_R) - 1.0
_CHUNK = 2184


def _overlaps(priors_ref, targets_ref, start, ch):
    pcx = priors_ref[pl.ds(start, ch), 0:1]
    pcy = priors_ref[pl.ds(start, ch), 1:2]
    pw = priors_ref[pl.ds(start, ch), 2:3]
    ph = priors_ref[pl.ds(start, ch), 3:4]
    px1 = pcx - pw / 2.0
    py1 = pcy - ph / 2.0
    px2 = pcx + pw / 2.0
    py2 = pcy + ph / 2.0

    tx1 = targets_ref[0, 0:1, :]
    ty1 = targets_ref[0, 1:2, :]
    tx2 = targets_ref[0, 2:3, :]
    ty2 = targets_ref[0, 3:4, :]

    iw = jnp.clip(jnp.minimum(tx2, px2) - jnp.maximum(tx1, px1), 0.0, None)
    ih = jnp.clip(jnp.minimum(ty2, py2) - jnp.maximum(ty1, py1), 0.0, None)
    inter = iw * ih
    area_t = (tx2 - tx1) * (ty2 - ty1)
    area_p = (px2 - px1) * (py2 - py1)
    return inter / (area_t + area_p - inter)


def _image_kernel(priors_ref, targets_ref, loc_ref, conf_ref,
                  loc_sum_ref, pos_cnt_ref, focal_ref, ign_cnt_ref):
    P = priors_ref.shape[0]
    n_obj = targets_ref.shape[2]
    big = jnp.int32(2 ** 30)
    chunks = [(s, min(_CHUNK, P - s)) for s in range(0, P, _CHUNK)]

    bpo = jnp.full((1, n_obj), -1.0, jnp.float32)
    bpi = jnp.zeros((1, n_obj), jnp.int32)
    for start, ch in chunks:
        ov = _overlaps(priors_ref, targets_ref, start, ch)
        c_bpo = jnp.max(ov, axis=0, keepdims=True)
        iota_p = jax.lax.broadcasted_iota(jnp.int32, (ch, n_obj), 0) + start
        c_bpi = jnp.min(jnp.where(ov == c_bpo, iota_p, big), axis=0,
                        keepdims=True)
        better = c_bpo > bpo
        bpi = jnp.where(better, c_bpi, bpi)
        bpo = jnp.maximum(bpo, c_bpo)

    tx1 = targets_ref[0, 0:1, :]
    ty1 = targets_ref[0, 1:2, :]
    tx2 = targets_ref[0, 2:3, :]
    ty2 = targets_ref[0, 3:4, :]
    tlab = targets_ref[0, 4:5, :]

    loc_sum = jnp.zeros((1, 1), jnp.float32)
    pos_cnt = jnp.zeros((1, 1), jnp.float32)
    ign_cnt = jnp.zeros((1, 1), jnp.float32)
    focal_sum = jnp.zeros((1, 1), jnp.float32)

    for start, ch in chunks:
        ov = _overlaps(priors_ref, targets_ref, start, ch)
        iota_t = jax.lax.broadcasted_iota(jnp.int32, (ch, n_obj), 1)
        iota_p = jax.lax.broadcasted_iota(jnp.int32, (ch, n_obj), 0) + start

        bto = jnp.max(ov, axis=1, keepdims=True)
        bti = jnp.min(jnp.where(ov == bto, iota_t, big), axis=1,
                      keepdims=True)
        forced_t = jnp.max(jnp.where(iota_p == bpi, iota_t, -1),
                           axis=1, keepdims=True)
        forced = forced_t >= 0
        bto = jnp.where(forced, 2.0, bto)
        bti = jnp.where(forced, forced_t, bti)

        eq = (bti == iota_t).astype(jnp.float32)
        mx1 = jnp.sum(eq * tx1, axis=1, keepdims=True)
        my1 = jnp.sum(eq * ty1, axis=1, keepdims=True)
        mx2 = jnp.sum(eq * tx2, axis=1, keepdims=True)
        my2 = jnp.sum(eq * ty2, axis=1, keepdims=True)
        mlab = jnp.sum(eq * tlab, axis=1, keepdims=True)

        pos = bto >= 0.5
        ign = jnp.logical_and(bto >= 0.4, bto < 0.5)
        posf = pos.astype(jnp.float32)
        ignf = ign.astype(jnp.float32)
        cls = jnp.where(pos | ign, mlab, -1.0).astype(jnp.int32)

        pcx = priors_ref[pl.ds(start, ch), 0:1]
        pcy = priors_ref[pl.ds(start, ch), 1:2]
        pw = priors_ref[pl.ds(start, ch), 2:3]
        ph = priors_ref[pl.ds(start, ch), 3:4]
        gcx = ((mx1 + mx2) / 2.0 - pcx) / (VAR0 * pw)
        gcy = ((my1 + my2) / 2.0 - pcy) / (VAR0 * ph)
        gw = jnp.log((mx2 - mx1) / pw) / VAR1
        gh = jnp.log((my2 - my1) / ph) / VAR1
        for c, g in enumerate((gcx, gcy, gw, gh)):
            d = jnp.abs(loc_ref[0, pl.ds(start, ch), c:c + 1] - g)
            small = (ALPHA_R / _B * (_B * d + 1.0)
                     * jnp.log(_B * d / BETA_R + 1.0) - ALPHA_R * d)
            large = GAMMA_R * d + GAMMA_R / _B - ALPHA_R * BETA_R
            bl = jnp.where(d < BETA_R, small, large)
            loc_sum += jnp.sum(bl * posf, axis=0, keepdims=True)
        pos_cnt += jnp.sum(posf, axis=0, keepdims=True)
        ign_cnt += jnp.sum(ignf, axis=0, keepdims=True)

        x = conf_ref[0, pl.ds(start, ch), :]
        u = jnp.exp(-jnp.abs(x))
        sp = jnp.maximum(x, 0.0) + jnp.log1p(u)
        sig = jnp.where(x >= 0.0, 1.0 / (1.0 + u), u / (1.0 + u))
        focal_sum += jnp.sum((1.0 - ALPHA_F) * sp * sig, axis=(0, 1),
                             keepdims=True)

        iota_c = jax.lax.broadcasted_iota(jnp.int32, x.shape, 1)
        xc = jnp.sum(jnp.where(iota_c == cls, x, 0.0), axis=1, keepdims=True)
        uc = jnp.exp(-jnp.abs(xc))
        spc = jnp.maximum(xc, 0.0) + jnp.log1p(uc)
        sigc = jnp.where(xc >= 0.0, 1.0 / (1.0 + uc), uc / (1.0 + uc))
        f0c = (1.0 - ALPHA_F) * spc * sigc
        f1c = ALPHA_F * (spc - xc) * (1.0 - sigc)
        focal_sum += (jnp.sum(posf * (f1c - f0c), axis=0, keepdims=True)
                      - jnp.sum(ignf * f0c, axis=0, keepdims=True))

    loc_sum_ref[0] = loc_sum
    pos_cnt_ref[0] = pos_cnt
    ign_cnt_ref[0] = ign_cnt
    focal_ref[0] = focal_sum


@jax.jit
def kernel(loc_data, conf_data, priors, targets):
    num, num_priors, num_classes = conf_data.shape
    n_obj = targets.shape[1]
    targets_t = jnp.transpose(targets, (0, 2, 1))

    out_sd = jax.ShapeDtypeStruct((num, 1, 1), jnp.float32)
    loc_sum, pos_cnt, focal, ign_cnt = pl.pallas_call(
        _image_kernel,
        grid=(num,),
        in_specs=[
            pl.BlockSpec((num_priors, 4), lambda i: (0, 0)),
            pl.BlockSpec((1, 5, n_obj), lambda i: (i, 0, 0)),
            pl.BlockSpec((1, num_priors, 4), lambda i: (i, 0, 0)),
            pl.BlockSpec((1, num_priors, num_classes), lambda i: (i, 0, 0)),
        ],
        out_specs=[pl.BlockSpec((1, 1, 1), lambda i: (i, 0, 0))] * 4,
        out_shape=[out_sd] * 4,
        compiler_params=pltpu.CompilerParams(
            dimension_semantics=("arbitrary",),
        ),
    )(priors, targets_t, loc_data, conf_data)

    loc_total = jnp.sum(loc_sum)
    pos_total = jnp.sum(pos_cnt)
    ign_total = jnp.sum(ign_cnt)
    focal_total = jnp.sum(focal)

    loss_l = loc_total / (4.0 * pos_total)
    denom = jnp.float32(num * num_priors * num_classes) - ign_total
    loss_c = focal_total / denom
    return (loss_l, loss_c)
